# baseline (speedup 1.0000x reference)
import functools

import jax
import jax.numpy as jnp
from jax import lax
from jax.experimental import pallas as pl
from jax.experimental.pallas import tpu as pltpu
from jax.experimental.pallas import tpu_sc as plsc

N_CORES = 2
N_SUBCORES = 16
N_WORKERS = N_CORES * N_SUBCORES
LANES = 16
BATCH = 16384
DIM = 32
BPW = BATCH // N_WORKERS
IDX_CHUNK = 128
N_IDX_CHUNKS = BPW // IDX_CHUNK
N_ROW_CHUNKS = BPW // LANES


def _mf_body(user_idx_hbm, item_idx_hbm, user_emb_hbm, item_emb_hbm,
             user_bias_hbm, item_bias_hbm, out_hbm,
             uidx_v, iidx_v, urows_v, irows_v, ubias_v, ibias_v, out_v,
             sem):
    c = lax.axis_index("c")
    s = lax.axis_index("s")
    wid = s * N_CORES + c
    base = wid * BPW

    for j in range(N_IDX_CHUNKS):
        pltpu.sync_copy(user_idx_hbm.at[pl.ds(base + j * IDX_CHUNK, IDX_CHUNK)],
                        uidx_v.at[j])
        pltpu.sync_copy(item_idx_hbm.at[pl.ds(base + j * IDX_CHUNK, IDX_CHUNK)],
                        iidx_v.at[j])

    copies = []
    for j in range(N_IDX_CHUNKS):
        sl = pl.ds(j * IDX_CHUNK, IDX_CHUNK)
        copies.append(pltpu.async_copy(
            user_emb_hbm.at[uidx_v.at[j]], urows_v.at[sl], sem))
        copies.append(pltpu.async_copy(
            item_emb_hbm.at[iidx_v.at[j]], irows_v.at[sl], sem))
        copies.append(pltpu.async_copy(
            user_bias_hbm.at[uidx_v.at[j]], ubias_v.at[sl], sem))
        copies.append(pltpu.async_copy(
            item_bias_hbm.at[iidx_v.at[j]], ibias_v.at[sl], sem))
    for cp in copies:
        cp.wait()

    lane = lax.iota(jnp.int32, LANES)

    dnums = lax.GatherDimensionNumbers(
        offset_dims=(), collapsed_slice_dims=(0,), start_index_map=(0,))

    def shuffle(x, perm):
        return lax.gather(x, perm[:, None], dnums, (1,),
                          mode=lax.GatherScatterMode.PROMISE_IN_BOUNDS)

    perms = [lane ^ m for m in (8, 4, 2, 1)]

    def chunk(k, carry):
        res = jnp.zeros((LANES,), jnp.float32)
        base_r = k * LANES
        for i in range(LANES):
            r = base_r + i
            u1 = urows_v[r, pl.ds(0, LANES)]
            u2 = urows_v[r, pl.ds(LANES, LANES)]
            v1 = irows_v[r, pl.ds(0, LANES)]
            v2 = irows_v[r, pl.ds(LANES, LANES)]
            s = u1 * v1 + u2 * v2
            for p in perms:
                s = s + shuffle(s, p)
            res = jnp.where(lane == i, s, res)
        x = res + ubias_v[pl.ds(base_r, LANES)] + ibias_v[pl.ds(base_r, LANES)]
        out_v[pl.ds(base_r, LANES)] = 1.0 / (1.0 + jnp.exp(-x))
        return carry

    lax.fori_loop(0, N_ROW_CHUNKS, chunk, 0)

    pltpu.sync_copy(out_v, out_hbm.at[pl.ds(base, BPW)])


@functools.partial(jax.jit, static_argnames=())
def kernel(user_idx, item_idx, user_emb, item_emb, user_bias, item_bias):
    mesh = plsc.VectorSubcoreMesh(core_axis_name="c", subcore_axis_name="s")
    f = pl.kernel(
        _mf_body,
        out_type=jax.ShapeDtypeStruct((BATCH,), jnp.float32),
        mesh=mesh,
        scratch_types=[
            pltpu.VMEM((N_IDX_CHUNKS, IDX_CHUNK), jnp.int32),
            pltpu.VMEM((N_IDX_CHUNKS, IDX_CHUNK), jnp.int32),
            pltpu.VMEM((BPW, DIM), jnp.float32),
            pltpu.VMEM((BPW, DIM), jnp.float32),
            pltpu.VMEM((BPW,), jnp.float32),
            pltpu.VMEM((BPW,), jnp.float32),
            pltpu.VMEM((BPW,), jnp.float32),
            pltpu.SemaphoreType.DMA,
        ],
        compiler_params=pltpu.CompilerParams(use_tc_tiling_on_sc=False),
    )
    return f(user_idx.astype(jnp.int32), item_idx.astype(jnp.int32),
             user_emb, item_emb,
             user_bias.reshape(-1), item_bias.reshape(-1))


# speedup vs baseline: 2.2290x; 2.2290x over previous
import functools

import jax
import jax.numpy as jnp
from jax import lax
from jax.experimental import pallas as pl
from jax.experimental.pallas import tpu as pltpu
from jax.experimental.pallas import tpu_sc as plsc

N_ROWS = 1_000_000
DIM = 32
BATCH = 16384
LANES = 16
N_WORKERS = 32
BPW = BATCH // N_WORKERS
RING = 8
HBPW = BPW // 4
NGROUPS = HBPW // RING
LAST_COL = (N_ROWS // 128 - 1) * 128
TAIL_START = (N_ROWS // 128) * 128
TAIL_N = N_ROWS - TAIL_START
HALF = 256


def _body(uT, iT, uidx_h, iidx_h, ubias_h, ibias_h, tailu_h, taili_h,
          out_h, urows_h, irows_h,
          uidx_v, iidx_v, wu, wi, bwu, bwi, rows_u, rows_i, bias_all,
          out_v, tailu_v, taili_v,
          sem_w, sem_c):
    c = lax.axis_index("c")
    s = lax.axis_index("s")
    w = s * 2 + c
    base = w * BPW

    pltpu.sync_copy(uidx_h.at[pl.ds(base, BPW)], uidx_v.at[pl.ds(0, BPW)])
    pltpu.sync_copy(iidx_h.at[pl.ds(base, BPW)], iidx_v.at[pl.ds(0, BPW)])
    pltpu.sync_copy(tailu_h, tailu_v)
    pltpu.sync_copy(taili_h, taili_v)

    lane = lax.iota(jnp.int32, LANES)
    dnums = lax.GatherDimensionNumbers(
        offset_dims=(), collapsed_slice_dims=(0,), start_index_map=(0,))

    def shuffle(x, perm):
        return lax.gather(x, perm[:, None], dnums, (1,),
                          mode=lax.GatherScatterMode.PROMISE_IN_BOUNDS)

    def win_addrs(cv):
        coff = jnp.clip((cv >> 7) * 128, 0, jnp.int32(LAST_COL))
        ju = jnp.minimum(cv - coff, jnp.int32(127))
        boff = pl.multiple_of(jnp.clip(cv & ~jnp.int32(7), 0, jnp.int32(N_ROWS - LANES)), 8)
        jb = cv - boff
        return coff, ju, boff, jb

    def fire(cu, ci, b):
        ucoff, _, uboff, _ = win_addrs(cu)
        icoff, _, iboff, _ = win_addrs(ci)
        for tr in range(4):
            pltpu.async_copy(
                uT.at[pl.ds(8 * tr, 8), pl.ds(pl.multiple_of(ucoff, 128), 128)],
                wu.at[b].at[pl.ds(8 * tr, 8)], sem_w)
            pltpu.async_copy(
                iT.at[pl.ds(8 * tr, 8), pl.ds(pl.multiple_of(icoff, 128), 128)],
                wi.at[b].at[pl.ds(8 * tr, 8)], sem_w)
        pltpu.async_copy(ubias_h.at[pl.ds(uboff, LANES)], bwu.at[b], sem_w)
        pltpu.async_copy(ibias_h.at[pl.ds(iboff, LANES)], bwi.at[b], sem_w)

    def wait_windows(b):
        pltpu.make_async_copy(uT.at[:, pl.ds(0, 128)], wu.at[b], sem_w).wait()
        pltpu.make_async_copy(iT.at[:, pl.ds(0, 128)], wi.at[b], sem_w).wait()
        pltpu.make_async_copy(ubias_h.at[pl.ds(0, LANES)], bwu.at[b], sem_w).wait()
        pltpu.make_async_copy(ibias_h.at[pl.ds(0, LANES)], bwi.at[b], sem_w).wait()

    def compact(cu, ci, sl, b):
        _, ju, _, jbu = win_addrs(cu)
        _, ji, _, jbi = win_addrs(ci)
        pltpu.async_copy(wu.at[b].at[:, ju], urows_h.at[sl], sem_c)
        pltpu.async_copy(wi.at[b].at[:, ji], irows_h.at[sl], sem_c)
        bu16 = bwu[b]
        bi16 = bwi[b]
        bb = (shuffle(bu16, jnp.full((LANES,), jbu, jnp.int32))
              + shuffle(bi16, jnp.full((LANES,), jbi, jnp.int32)))
        return bb

    def wait_compact(b):
        pltpu.make_async_copy(wu.at[b].at[:, 0], urows_h.at[0], sem_c).wait()
        pltpu.make_async_copy(wi.at[b].at[:, 0], irows_h.at[0], sem_c).wait()

    perms = [lane ^ m for m in (8, 4, 2, 1)]
    NG16 = HBPW // LANES

    def chunk16(off):
        o = pl.multiple_of(off, LANES)
        return uidx_v[pl.ds(o, LANES)], iidx_v[pl.ds(o, LANES)]

    cu0, ci0 = chunk16(0)
    for b in range(RING):
        fire(cu0[b], ci0[b], b)

    NGALL = BPW // LANES

    def slot_dot(cu, ci, b):
        ucoff, juu, uboff, jbu = win_addrs(cu)
        icoff, jii, iboff, jbi = win_addrs(ci)
        jug = pl.multiple_of((juu // LANES) * LANES, LANES)
        jig = pl.multiple_of((jii // LANES) * LANES, LANES)
        lu = jnp.full((LANES,), juu - jug, jnp.int32)
        li = jnp.full((LANES,), jii - jig, jnp.int32)
        ut = cu >= TAIL_START
        it = ci >= TAIL_START
        rtu = jnp.clip(cu - jnp.int32(TAIL_START), 0, TAIL_N - 1)
        rti = jnp.clip(ci - jnp.int32(TAIL_START), 0, TAIL_N - 1)
        acc = jnp.zeros((LANES,), jnp.float32)
        for d in range(DIM):
            dblk = (d // LANES) * LANES
            ub = jnp.where(ut, tailu_v[rtu, pl.ds(dblk, LANES)],
                           wu[b, d, pl.ds(jug, LANES)])
            vb = jnp.where(it, taili_v[rti, pl.ds(dblk, LANES)],
                           wi[b, d, pl.ds(jig, LANES)])
            lu_d = jnp.where(ut, jnp.full((LANES,), d % LANES, jnp.int32), lu)
            li_d = jnp.where(it, jnp.full((LANES,), d % LANES, jnp.int32), li)
            bu_d = shuffle(ub, lu_d)
            bv_d = shuffle(vb, li_d)
            acc = acc + bu_d * bv_d
        bu = shuffle(bwu[b], jnp.full((LANES,), jbu, jnp.int32))
        bi = shuffle(bwi[b], jnp.full((LANES,), jbi, jnp.int32))
        return acc + bu + bi

    def halfstep(gg, sub, fire_next):
        cuA, ciA = chunk16(gg * LANES)
        res = jnp.zeros((LANES,), jnp.float32)
        for b in range(RING):
            wait_windows(b)
        for b in range(RING):
            ln = sub * RING + b
            x = slot_dot(cuA[ln], ciA[ln], b)
            res = jnp.where(lane == ln, x, res)
        if fire_next:
            if sub == 0:
                for b in range(RING):
                    fire(cuA[RING + b], ciA[RING + b], b)
            else:
                cuN, ciN = chunk16((gg + 1) * LANES)
                for b in range(RING):
                    fire(cuN[b], ciN[b], b)
        return res

    def body(gg, carry2):
        r0 = halfstep(gg, 0, True)
        r1 = halfstep(gg, 1, True)
        x = r0 + r1
        out_v[pl.ds(pl.multiple_of(gg * LANES, LANES), LANES)] = (
            1.0 / (1.0 + jnp.exp(-x)))
        return carry2

    lax.fori_loop(0, NGALL, body, 0)
    for b in range(RING):
        wait_windows(b)

    pltpu.sync_copy(out_v, out_h.at[pl.ds(base, BPW)])


def kernel(user_idx, item_idx, user_emb, item_emb, user_bias, item_bias):
    mesh = plsc.VectorSubcoreMesh(core_axis_name="c", subcore_axis_name="s")
    tail_u = jnp.pad(user_emb[TAIL_START:], ((0, 0), (0, 128 - DIM)))
    tail_i = jnp.pad(item_emb[TAIL_START:], ((0, 0), (0, 128 - DIM)))
    f = pl.kernel(
        _body,
        out_type=(jax.ShapeDtypeStruct((BATCH,), jnp.float32),
                  jax.ShapeDtypeStruct((BATCH, DIM), jnp.float32),
                  jax.ShapeDtypeStruct((BATCH, DIM), jnp.float32)),
        mesh=mesh,
        scratch_types=[
            pltpu.VMEM((BPW + LANES,), jnp.int32),
            pltpu.VMEM((BPW + LANES,), jnp.int32),
            pltpu.VMEM((RING, DIM, 128), jnp.float32),
            pltpu.VMEM((RING, DIM, 128), jnp.float32),
            pltpu.VMEM((RING, LANES), jnp.float32),
            pltpu.VMEM((RING, LANES), jnp.float32),
            pltpu.VMEM((HBPW, DIM), jnp.float32),
            pltpu.VMEM((HBPW, DIM), jnp.float32),
            pltpu.VMEM((BPW,), jnp.float32),
            pltpu.VMEM((BPW,), jnp.float32),
            pltpu.VMEM((TAIL_N, 128), jnp.float32),
            pltpu.VMEM((TAIL_N, 128), jnp.float32),
            pltpu.SemaphoreType.DMA,
            pltpu.SemaphoreType.DMA,
        ],
        compiler_params=pltpu.CompilerParams(use_tc_tiling_on_sc=True),
    )
    out, _, _ = f(user_emb.T, item_emb.T,
                  user_idx.astype(jnp.int32), item_idx.astype(jnp.int32),
                  user_bias.reshape(-1), item_bias.reshape(-1),
                  tail_u, tail_i)
    return out


# speedup vs baseline: 2.5296x; 1.1349x over previous
import functools

import jax
import jax.numpy as jnp
from jax import lax
from jax.experimental import pallas as pl
from jax.experimental.pallas import tpu as pltpu
from jax.experimental.pallas import tpu_sc as plsc

N_ROWS = 1_000_000
DIM = 32
BATCH = 16384
LANES = 16
N_WORKERS = 32
BPW = BATCH // N_WORKERS
RING = 8
HBPW = BPW // 4
NGROUPS = HBPW // RING
LAST_COL = (N_ROWS // 128 - 1) * 128
TAIL_START = (N_ROWS // 128) * 128
TAIL_N = N_ROWS - TAIL_START
HALF = 256


def _body(uT, iT, uidx_h, iidx_h, ubias_h, ibias_h, tailu_h, taili_h,
          out_h, urows_h, irows_h,
          uidx_v, iidx_v, wu, wi, bwu, bwi, rows_u, rows_i, bias_all,
          out_v, tailu_v, taili_v,
          sem_w, sem_c):
    c = lax.axis_index("c")
    s = lax.axis_index("s")
    w = s * 2 + c
    base = w * BPW

    pltpu.sync_copy(uidx_h.at[pl.ds(base, BPW)], uidx_v.at[pl.ds(0, BPW)])
    pltpu.sync_copy(iidx_h.at[pl.ds(base, BPW)], iidx_v.at[pl.ds(0, BPW)])
    pltpu.sync_copy(tailu_h, tailu_v)
    pltpu.sync_copy(taili_h, taili_v)

    lane = lax.iota(jnp.int32, LANES)
    dnums = lax.GatherDimensionNumbers(
        offset_dims=(), collapsed_slice_dims=(0,), start_index_map=(0,))

    def shuffle(x, perm):
        return lax.gather(x, perm[:, None], dnums, (1,),
                          mode=lax.GatherScatterMode.PROMISE_IN_BOUNDS)

    def win_addrs(cv):
        coff = jnp.clip((cv >> 7) * 128, 0, jnp.int32(LAST_COL))
        ju = jnp.minimum(cv - coff, jnp.int32(127))
        boff = pl.multiple_of(jnp.clip(cv & ~jnp.int32(7), 0, jnp.int32(N_ROWS - LANES)), 8)
        jb = cv - boff
        return coff, ju, boff, jb

    def fire(cu, ci, b):
        ucoff, _, uboff, _ = win_addrs(cu)
        icoff, _, iboff, _ = win_addrs(ci)
        for tr in range(4):
            pltpu.async_copy(
                uT.at[pl.ds(8 * tr, 8), pl.ds(pl.multiple_of(ucoff, 128), 128)],
                wu.at[b].at[pl.ds(8 * tr, 8)], sem_w)
            pltpu.async_copy(
                iT.at[pl.ds(8 * tr, 8), pl.ds(pl.multiple_of(icoff, 128), 128)],
                wi.at[b].at[pl.ds(8 * tr, 8)], sem_w)
        pltpu.async_copy(ubias_h.at[pl.ds(uboff, LANES)], bwu.at[b], sem_w)
        pltpu.async_copy(ibias_h.at[pl.ds(iboff, LANES)], bwi.at[b], sem_w)

    def wait_windows(b):
        pltpu.make_async_copy(uT.at[:, pl.ds(0, 128)], wu.at[b], sem_w).wait()
        pltpu.make_async_copy(iT.at[:, pl.ds(0, 128)], wi.at[b], sem_w).wait()
        pltpu.make_async_copy(ubias_h.at[pl.ds(0, LANES)], bwu.at[b], sem_w).wait()
        pltpu.make_async_copy(ibias_h.at[pl.ds(0, LANES)], bwi.at[b], sem_w).wait()

    def compact(cu, ci, sl, b):
        _, ju, _, jbu = win_addrs(cu)
        _, ji, _, jbi = win_addrs(ci)
        pltpu.async_copy(wu.at[b].at[:, ju], urows_h.at[sl], sem_c)
        pltpu.async_copy(wi.at[b].at[:, ji], irows_h.at[sl], sem_c)
        bu16 = bwu[b]
        bi16 = bwi[b]
        bb = (shuffle(bu16, jnp.full((LANES,), jbu, jnp.int32))
              + shuffle(bi16, jnp.full((LANES,), jbi, jnp.int32)))
        return bb

    def wait_compact(b):
        pltpu.make_async_copy(wu.at[b].at[:, 0], urows_h.at[0], sem_c).wait()
        pltpu.make_async_copy(wi.at[b].at[:, 0], irows_h.at[0], sem_c).wait()

    perms = [lane ^ m for m in (8, 4, 2, 1)]
    NG16 = HBPW // LANES

    def chunk16(off):
        o = pl.multiple_of(off, LANES)
        return uidx_v[pl.ds(o, LANES)], iidx_v[pl.ds(o, LANES)]

    cu0, ci0 = chunk16(0)
    for b in range(RING):
        fire(cu0[b], ci0[b], b)

    NGALL = BPW // LANES

    def slot_dot(cu, ci, b):
        ucoff, juu, uboff, jbu = win_addrs(cu)
        icoff, jii, iboff, jbi = win_addrs(ci)
        jug = pl.multiple_of((juu // LANES) * LANES, LANES)
        jig = pl.multiple_of((jii // LANES) * LANES, LANES)
        lu = jnp.full((LANES,), juu - jug, jnp.int32)
        li = jnp.full((LANES,), jii - jig, jnp.int32)
        ut = cu >= TAIL_START
        it = ci >= TAIL_START
        rtu = jnp.clip(cu - jnp.int32(TAIL_START), 0, TAIL_N - 1)
        rti = jnp.clip(ci - jnp.int32(TAIL_START), 0, TAIL_N - 1)
        acc = jnp.zeros((LANES,), jnp.float32)
        for d in range(DIM):
            dblk = (d // LANES) * LANES
            ub = jnp.where(ut, tailu_v[rtu, pl.ds(dblk, LANES)],
                           wu[b, d, pl.ds(jug, LANES)])
            vb = jnp.where(it, taili_v[rti, pl.ds(dblk, LANES)],
                           wi[b, d, pl.ds(jig, LANES)])
            lu_d = jnp.where(ut, jnp.full((LANES,), d % LANES, jnp.int32), lu)
            li_d = jnp.where(it, jnp.full((LANES,), d % LANES, jnp.int32), li)
            bu_d = shuffle(ub, lu_d)
            bv_d = shuffle(vb, li_d)
            acc = acc + bu_d * bv_d
        bu = shuffle(bwu[b], jnp.full((LANES,), jbu, jnp.int32))
        bi = shuffle(bwi[b], jnp.full((LANES,), jbi, jnp.int32))
        return acc + bu + bi

    def halfstep(gg, sub, fire_next):
        cuA, ciA = chunk16(gg * LANES)
        if sub == 1:
            cuN, ciN = chunk16((gg + 1) * LANES)
        res = jnp.zeros((LANES,), jnp.float32)
        for b in range(RING):
            wait_windows(b)
        for b in range(RING):
            ln = sub * RING + b
            x = slot_dot(cuA[ln], ciA[ln], b)
            res = jnp.where(lane == ln, x, res)
            if fire_next:
                if sub == 0:
                    fire(cuA[RING + b], ciA[RING + b], b)
                else:
                    fire(cuN[b], ciN[b], b)
        return res

    def body(gg, carry2):
        r0 = halfstep(gg, 0, True)
        r1 = halfstep(gg, 1, True)
        x = r0 + r1
        out_v[pl.ds(pl.multiple_of(gg * LANES, LANES), LANES)] = (
            1.0 / (1.0 + jnp.exp(-x)))
        return carry2

    lax.fori_loop(0, NGALL, body, 0)
    for b in range(RING):
        wait_windows(b)

    pltpu.sync_copy(out_v, out_h.at[pl.ds(base, BPW)])


def kernel(user_idx, item_idx, user_emb, item_emb, user_bias, item_bias):
    mesh = plsc.VectorSubcoreMesh(core_axis_name="c", subcore_axis_name="s")
    tail_u = jnp.pad(user_emb[TAIL_START:], ((0, 0), (0, 128 - DIM)))
    tail_i = jnp.pad(item_emb[TAIL_START:], ((0, 0), (0, 128 - DIM)))
    f = pl.kernel(
        _body,
        out_type=(jax.ShapeDtypeStruct((BATCH,), jnp.float32),
                  jax.ShapeDtypeStruct((BATCH, DIM), jnp.float32),
                  jax.ShapeDtypeStruct((BATCH, DIM), jnp.float32)),
        mesh=mesh,
        scratch_types=[
            pltpu.VMEM((BPW + LANES,), jnp.int32),
            pltpu.VMEM((BPW + LANES,), jnp.int32),
            pltpu.VMEM((RING, DIM, 128), jnp.float32),
            pltpu.VMEM((RING, DIM, 128), jnp.float32),
            pltpu.VMEM((RING, LANES), jnp.float32),
            pltpu.VMEM((RING, LANES), jnp.float32),
            pltpu.VMEM((HBPW, DIM), jnp.float32),
            pltpu.VMEM((HBPW, DIM), jnp.float32),
            pltpu.VMEM((BPW,), jnp.float32),
            pltpu.VMEM((BPW,), jnp.float32),
            pltpu.VMEM((TAIL_N, 128), jnp.float32),
            pltpu.VMEM((TAIL_N, 128), jnp.float32),
            pltpu.SemaphoreType.DMA,
            pltpu.SemaphoreType.DMA,
        ],
        compiler_params=pltpu.CompilerParams(use_tc_tiling_on_sc=True),
    )
    out, _, _ = f(user_emb.T, item_emb.T,
                  user_idx.astype(jnp.int32), item_idx.astype(jnp.int32),
                  user_bias.reshape(-1), item_bias.reshape(-1),
                  tail_u, tail_i)
    return out
